# baseline (speedup 1.0000x reference)
import jax
import jax.numpy as jnp
from jax import lax
from jax.experimental import pallas as pl
from jax.experimental.pallas import tpu as pltpu


def _make_kernel(N, nhid, BR, NBUF):
    NBLK = N // BR

    def _gcn_kernel(
        w_ref, b_ref, x_ref, adj_ref, out_ref, xbuf_ref, buf_ref, support_ref, sems
    ):
        def copy_in(blk, slot):
            return pltpu.make_async_copy(
                adj_ref.at[pl.ds(blk * BR, BR), :],
                buf_ref.at[slot],
                sems.at[slot],
            )

        x_copy = pltpu.make_async_copy(x_ref, xbuf_ref, sems.at[NBUF])

        for slot in range(NBUF):
            copy_in(slot, slot).start()
        x_copy.start()
        x_copy.wait()
        support_ref[...] = jnp.dot(
            xbuf_ref[...], w_ref[...], preferred_element_type=jnp.float32
        )

        def step(blk, slot, issue_next):
            copy_in(blk, slot).wait()
            out = jnp.dot(
                buf_ref[slot], support_ref[...], preferred_element_type=jnp.float32
            )
            if issue_next:

                @pl.when(blk + NBUF < NBLK)
                def _():
                    copy_in(blk + NBUF, slot).start()

            h = jnp.maximum(out + b_ref[...], 0.0)
            m = jnp.max(h, axis=1, keepdims=True)
            s = h - m
            lse = jnp.log(jnp.sum(jnp.exp(s), axis=1, keepdims=True))
            out_ref[pl.ds(blk * BR, BR), :] = s - lse

        def outer(j, carry):
            for slot in range(NBUF):
                step(j * NBUF + slot, slot, True)
            return carry

        lax.fori_loop(0, NBLK // NBUF, outer, 0, unroll=False)
        for tail in range((NBLK // NBUF) * NBUF, NBLK):
            step(tail, tail % NBUF, False)

    return _gcn_kernel


def kernel(x, adj, W, b):
    N, nfeat = x.shape
    nhid = W.shape[1]
    BR = 400
    NBUF = 3

    return pl.pallas_call(
        _make_kernel(N, nhid, BR, NBUF),
        in_specs=[
            pl.BlockSpec(memory_space=pltpu.MemorySpace.VMEM),
            pl.BlockSpec(memory_space=pltpu.MemorySpace.VMEM),
            pl.BlockSpec(memory_space=pltpu.MemorySpace.HBM),
            pl.BlockSpec(memory_space=pltpu.MemorySpace.HBM),
        ],
        out_specs=pl.BlockSpec(memory_space=pltpu.MemorySpace.VMEM),
        out_shape=jax.ShapeDtypeStruct((N, nhid), jnp.float32),
        scratch_shapes=[
            pltpu.VMEM((N, nfeat), jnp.float32),
            pltpu.VMEM((NBUF, BR, N), jnp.float32),
            pltpu.VMEM((N, nhid), jnp.float32),
            pltpu.SemaphoreType.DMA((NBUF + 1,)),
        ],
        compiler_params=pltpu.CompilerParams(
            vmem_limit_bytes=100 * 1024 * 1024,
        ),
    )(W, b.reshape(1, nhid), x, adj)


# speedup vs baseline: 1.0532x; 1.0532x over previous
import jax
import jax.numpy as jnp
from jax import lax
from jax.experimental import pallas as pl
from jax.experimental.pallas import tpu as pltpu


def _make_kernel(N, nhid, BR, NBUF):
    NBLK = N // BR

    def _gcn_kernel(
        w_ref, b_ref, x_ref, adj_ref, out_ref, xbuf_ref, buf_ref, support_ref, sems
    ):
        def copy_in(blk, slot):
            return pltpu.make_async_copy(
                adj_ref.at[pl.ds(blk * BR, BR), :],
                buf_ref.at[slot],
                sems.at[slot],
            )

        x_copy = pltpu.make_async_copy(x_ref, xbuf_ref, sems.at[NBUF])

        x_copy.start()
        for slot in range(NBUF):
            copy_in(slot, slot).start()
        x_copy.wait()
        support_ref[...] = jnp.dot(
            xbuf_ref[...], w_ref[...], preferred_element_type=jnp.float32
        )

        def step(blk, slot, issue_next):
            copy_in(blk, slot).wait()
            out = jnp.dot(
                buf_ref[slot], support_ref[...], preferred_element_type=jnp.float32
            )
            if issue_next:

                @pl.when(blk + NBUF < NBLK)
                def _():
                    copy_in(blk + NBUF, slot).start()

            h = jnp.maximum(out + b_ref[...], 0.0)
            m = jnp.max(h, axis=1, keepdims=True)
            s = h - m
            lse = jnp.log(jnp.sum(jnp.exp(s), axis=1, keepdims=True))
            out_ref[pl.ds(blk * BR, BR), :] = s - lse

        def outer(j, carry):
            for slot in range(NBUF):
                step(j * NBUF + slot, slot, True)
            return carry

        lax.fori_loop(0, NBLK // NBUF, outer, 0, unroll=False)
        for tail in range((NBLK // NBUF) * NBUF, NBLK):
            step(tail, tail % NBUF, False)

    return _gcn_kernel


def kernel(x, adj, W, b):
    N, nfeat = x.shape
    nhid = W.shape[1]
    BR = 400
    NBUF = 3

    return pl.pallas_call(
        _make_kernel(N, nhid, BR, NBUF),
        in_specs=[
            pl.BlockSpec(memory_space=pltpu.MemorySpace.VMEM),
            pl.BlockSpec(memory_space=pltpu.MemorySpace.VMEM),
            pl.BlockSpec(memory_space=pltpu.MemorySpace.HBM),
            pl.BlockSpec(memory_space=pltpu.MemorySpace.HBM),
        ],
        out_specs=pl.BlockSpec(memory_space=pltpu.MemorySpace.VMEM),
        out_shape=jax.ShapeDtypeStruct((N, nhid), jnp.float32),
        scratch_shapes=[
            pltpu.VMEM((N, nfeat), jnp.float32),
            pltpu.VMEM((NBUF, BR, N), jnp.float32),
            pltpu.VMEM((N, nhid), jnp.float32),
            pltpu.SemaphoreType.DMA((NBUF + 1,)),
        ],
        compiler_params=pltpu.CompilerParams(
            vmem_limit_bytes=100 * 1024 * 1024,
        ),
    )(W, b.reshape(1, nhid), x, adj)
